# baseline (speedup 1.0000x reference)
import functools

import jax
import jax.numpy as jnp
from jax import lax
from jax.experimental import pallas as pl
from jax.experimental.pallas import tpu as pltpu
from jax.experimental.pallas import tpu_sc as plsc

EPS = 1e-7


def _clipnorm(v):
    return jnp.clip(
        jnp.sqrt(jnp.sum(v * v, axis=-1, keepdims=True)), 1e-10, None)


def _artanh(v):
    v = jnp.clip(v, -1.0 + EPS, 1.0 - EPS)
    return 0.5 * jnp.log((1.0 + v) / (1.0 - v))


def _mobius_add(x, y):
    x2 = jnp.sum(x * x, axis=-1, keepdims=True)
    y2 = jnp.sum(y * y, axis=-1, keepdims=True)
    xy = jnp.sum(x * y, axis=-1, keepdims=True)
    num = (1.0 + 2.0 * xy + y2) * x + (1.0 - x2) * y
    den = 1.0 + 2.0 * xy + x2 * y2
    return num / jnp.clip(den, 1e-10, None)


def _table_body(x_ref, w_ref, out0_ref, out1_ref, *, n_rel, half):
    x = x_ref[...]
    xn = _clipnorm(x)
    node = x / xn * _artanh(xn)
    for r in range(n_rel):
        y = jnp.dot(node, w_ref[r], preferred_element_type=jnp.float32)
        yn = _clipnorm(y)
        z = jnp.tanh(yn) * y / yn
        out0_ref[r] = z[:, :half]
        out1_ref[r] = z[:, half:]


def _build_table(x, w_bd, tile_n):
    n, d = x.shape
    n_rel = w_bd.shape[0]
    half = d // 2
    body = functools.partial(_table_body, n_rel=n_rel, half=half)
    out_shape = jax.ShapeDtypeStruct((n_rel, n, half), jnp.float32)
    return pl.pallas_call(
        body,
        grid=(n // tile_n,),
        in_specs=[
            pl.BlockSpec((tile_n, d), lambda i: (i, 0)),
            pl.BlockSpec((n_rel, d, d), lambda i: (0, 0, 0)),
        ],
        out_specs=[
            pl.BlockSpec((n_rel, tile_n, half), lambda i: (0, i, 0)),
            pl.BlockSpec((n_rel, tile_n, half), lambda i: (0, i, 0)),
        ],
        out_shape=[out_shape, out_shape],
    )(x, w_bd)


def _make_sc_scatter(n, ep, half, chunk):
    n_tiles = 16
    iters = ep // chunk
    n_acc = n + 16
    rows_pt = (n // n_tiles) // 8 * 8
    rem = n - n_tiles * rows_pt
    mesh = plsc.VectorSubcoreMesh(core_axis_name="c", subcore_axis_name="s")

    @functools.partial(
        pl.kernel,
        out_type=jax.ShapeDtypeStruct((2, n, half), jnp.float32),
        mesh=mesh,
        scratch_types=[
            pltpu.VMEM((2, chunk), jnp.int32),
            pltpu.VMEM((2, chunk), jnp.int32),
            pltpu.VMEM((chunk, half), jnp.float32),
            pltpu.VMEM((chunk, half), jnp.float32),
            pltpu.VMEM_SHARED((n_acc, half), jnp.float32),
            pltpu.SemaphoreType.DMA,
            pltpu.SemaphoreType.DMA,
            pltpu.SemaphoreType.DMA,
            pltpu.SemaphoreType.DMA,
        ],
    )
    def sc_scatter(t0_hbm, t1_hbm, eidx_hbm, zeros_hbm, out_hbm,
                   ibuf_a, ibuf_b, rows_a, rows_b, hacc,
                   sem_ia, sem_ib, sem_ga, sem_gb):
        c = lax.axis_index("c")
        s = lax.axis_index("s")
        row0 = s * rows_pt
        pltpu.sync_copy(zeros_hbm.at[pl.ds(row0, rows_pt)],
                        hacc.at[pl.ds(row0, rows_pt)])
        if rem:
            @pl.when(s == 0)
            def _():
                pltpu.sync_copy(zeros_hbm.at[pl.ds(n_tiles * rows_pt, rem)],
                                hacc.at[pl.ds(n_tiles * rows_pt, rem)])
        plsc.subcore_barrier()

        def run(tref):
            def ifetch(k, ibuf, sem):
                pltpu.async_copy(eidx_hbm.at[s, k], ibuf, sem)

            def iwait(ibuf, sem):
                pltpu.make_async_copy(eidx_hbm.at[s, 0], ibuf, sem).wait()

            def gather(ibuf, rows, sem):
                pltpu.async_copy(tref.at[ibuf.at[0]], rows, sem)

            def gwait(ibuf, rows, sem):
                pltpu.make_async_copy(tref.at[ibuf.at[0]], rows, sem).wait()

            def scatter(ibuf, rows):
                pltpu.sync_copy(rows, hacc.at[ibuf.at[1]], add=True)

            def step(k, cur, nxt):
                (ibuf_c, sem_ic, rows_c, sem_gc) = cur
                (ibuf_n, sem_in, rows_n, sem_gn) = nxt
                iwait(ibuf_n, sem_in)
                gather(ibuf_n, rows_n, sem_gn)
                gwait(ibuf_c, rows_c, sem_gc)
                scatter(ibuf_c, rows_c)
                ifetch(k + 2, ibuf_c, sem_ic)

            buf_a = (ibuf_a, sem_ia, rows_a, sem_ga)
            buf_b = (ibuf_b, sem_ib, rows_b, sem_gb)

            ifetch(0, ibuf_a, sem_ia)
            iwait(ibuf_a, sem_ia)
            gather(ibuf_a, rows_a, sem_ga)
            ifetch(1, ibuf_b, sem_ib)

            def body(i2, carry):
                step(i2 * 2, buf_a, buf_b)
                step(i2 * 2 + 1, buf_b, buf_a)
                return carry
            lax.fori_loop(0, iters // 2 - 1, body, 0)
            iwait(ibuf_b, sem_ib)
            gather(ibuf_b, rows_b, sem_gb)
            gwait(ibuf_a, rows_a, sem_ga)
            scatter(ibuf_a, rows_a)
            gwait(ibuf_b, rows_b, sem_gb)
            scatter(ibuf_b, rows_b)

        @pl.when(c == 0)
        def _():
            run(t0_hbm)

        @pl.when(c == 1)
        def _():
            run(t1_hbm)

        plsc.subcore_barrier()
        pltpu.sync_copy(hacc.at[pl.ds(row0, rows_pt)],
                        out_hbm.at[c, pl.ds(row0, rows_pt)])
        if rem:
            @pl.when(s == 0)
            def _():
                pltpu.sync_copy(hacc.at[pl.ds(n_tiles * rows_pt, rem)],
                                out_hbm.at[c, pl.ds(n_tiles * rows_pt, rem)])

    return sc_scatter


def _combine_body(x_ref, h0_ref, h1_ref, nrm_ref, lwt_ref, b_ref, o_ref):
    x = x_ref[...]
    mx = jnp.dot(x, lwt_ref[...], preferred_element_type=jnp.float32)
    xn = _clipnorm(x)
    mxn = _clipnorm(mx)
    loop_msg = jnp.tanh(mxn / xn * _artanh(xn)) * mx / mxn
    h = jnp.concatenate([h0_ref[...], h1_ref[...]], axis=-1) * nrm_ref[...]
    h = _mobius_add(h, b_ref[...])
    h = _mobius_add(h, loop_msg)
    o_ref[...] = h


def _combine(x, h0, h1, norm, lw_t, bias_row, tile_n):
    n, d = x.shape
    half = d // 2
    return pl.pallas_call(
        _combine_body,
        grid=(n // tile_n,),
        in_specs=[
            pl.BlockSpec((tile_n, d), lambda i: (i, 0)),
            pl.BlockSpec((tile_n, half), lambda i: (i, 0)),
            pl.BlockSpec((tile_n, half), lambda i: (i, 0)),
            pl.BlockSpec((tile_n, 1), lambda i: (i, 0)),
            pl.BlockSpec((d, d), lambda i: (0, 0)),
            pl.BlockSpec((1, d), lambda i: (0, 0)),
        ],
        out_specs=pl.BlockSpec((tile_n, d), lambda i: (i, 0)),
        out_shape=jax.ShapeDtypeStruct((n, d), jnp.float32),
    )(x, h0, h1, norm, lw_t, bias_row)


def kernel(x, edge_index, edge_type, norm, weight, loop_weight, bias):
    n, d = x.shape
    e = edge_index.shape[1]
    n_rel = weight.shape[0]
    n_blk = 4
    sub = d // n_blk
    half = d // 2

    src = edge_index[0].astype(jnp.int32)
    dst = edge_index[1].astype(jnp.int32)
    et = edge_type.astype(jnp.int32)
    gidx = et * n + src
    order = jnp.argsort((dst >= n // 2).astype(jnp.int32), stable=True)
    gidx = gidx[order]
    dst = dst[order]

    w4 = weight.reshape(n_rel, n_blk, sub, sub)
    eye = jnp.eye(n_blk, dtype=jnp.float32)
    w_bd = (w4[:, :, :, None, :] *
            eye[None, :, None, :, None]).reshape(n_rel, d, d)

    t0, t1 = _build_table(x, w_bd, tile_n=400)
    t0 = t0.reshape(n_rel * n, half)
    t1 = t1.reshape(n_rel * n, half)

    chunk = 128
    ep_raw = e // 16
    ep = -(-ep_raw // (2 * chunk)) * (2 * chunk)
    iters = ep // chunk
    pad = ep - ep_raw
    gidx_t = jnp.pad(gidx.reshape(16, ep_raw), ((0, 0), (0, pad)))
    dst_t = jnp.pad(dst.reshape(16, ep_raw), ((0, 0), (0, pad)),
                    constant_values=n)
    eidx = jnp.stack([gidx_t.reshape(16, iters, chunk),
                      dst_t.reshape(16, iters, chunk)], axis=2)
    zeros = jnp.zeros((n, half), jnp.float32)
    h2 = _make_sc_scatter(n, ep, half, chunk)(t0, t1, eidx, zeros)

    return _combine(x, h2[0], h2[1], norm, loop_weight.T,
                    bias.reshape(1, d), tile_n=400)


# speedup vs baseline: 1.5412x; 1.5412x over previous
import functools

import jax
import jax.numpy as jnp
from jax import lax
from jax.experimental import pallas as pl
from jax.experimental.pallas import tpu as pltpu
from jax.experimental.pallas import tpu_sc as plsc

EPS = 1e-7


def _clipnorm(v):
    return jnp.clip(
        jnp.sqrt(jnp.sum(v * v, axis=-1, keepdims=True)), 1e-10, None)


def _artanh(v):
    v = jnp.clip(v, -1.0 + EPS, 1.0 - EPS)
    return 0.5 * jnp.log((1.0 + v) / (1.0 - v))


def _mobius_add(x, y):
    x2 = jnp.sum(x * x, axis=-1, keepdims=True)
    y2 = jnp.sum(y * y, axis=-1, keepdims=True)
    xy = jnp.sum(x * y, axis=-1, keepdims=True)
    num = (1.0 + 2.0 * xy + y2) * x + (1.0 - x2) * y
    den = 1.0 + 2.0 * xy + x2 * y2
    return num / jnp.clip(den, 1e-10, None)


def _table_body(x_ref, w_ref, out0_ref, out1_ref, *, n_rel, half):
    x = x_ref[...]
    xn = _clipnorm(x)
    node = x / xn * _artanh(xn)
    for r in range(n_rel):
        y = jnp.dot(node, w_ref[r], preferred_element_type=jnp.float32)
        yn = _clipnorm(y)
        z = jnp.tanh(yn) * y / yn
        out0_ref[r] = z[:, :half]
        out1_ref[r] = z[:, half:]


def _build_table(x, w_bd, tile_n):
    n, d = x.shape
    n_rel = w_bd.shape[0]
    half = d // 2
    body = functools.partial(_table_body, n_rel=n_rel, half=half)
    out_shape = jax.ShapeDtypeStruct((n_rel, n, half), jnp.float32)
    return pl.pallas_call(
        body,
        grid=(n // tile_n,),
        in_specs=[
            pl.BlockSpec((tile_n, d), lambda i: (i, 0)),
            pl.BlockSpec((n_rel, d, d), lambda i: (0, 0, 0)),
        ],
        out_specs=[
            pl.BlockSpec((n_rel, tile_n, half), lambda i: (0, i, 0)),
            pl.BlockSpec((n_rel, tile_n, half), lambda i: (0, i, 0)),
        ],
        out_shape=[out_shape, out_shape],
    )(x, w_bd)


def _make_sc_scatter(n, ep, half, chunk):
    n_tiles = 16
    iters = ep // chunk
    n_acc = n + 16
    rows_pt = (n // n_tiles) // 8 * 8
    rem = n - n_tiles * rows_pt
    mesh = plsc.VectorSubcoreMesh(core_axis_name="c", subcore_axis_name="s")

    @functools.partial(
        pl.kernel,
        out_type=jax.ShapeDtypeStruct((2, n, half), jnp.float32),
        mesh=mesh,
        scratch_types=[
            pltpu.VMEM((2, chunk), jnp.int32),
            pltpu.VMEM((2, chunk), jnp.int32),
            pltpu.VMEM((chunk, half), jnp.float32),
            pltpu.VMEM((chunk, half), jnp.float32),
            pltpu.VMEM_SHARED((n_acc, half), jnp.float32),
            pltpu.SemaphoreType.DMA,
            pltpu.SemaphoreType.DMA,
            pltpu.SemaphoreType.DMA,
            pltpu.SemaphoreType.DMA,
        ],
    )
    def sc_scatter(t0_hbm, t1_hbm, eidx_hbm, zeros_hbm, out_hbm,
                   ibuf_a, ibuf_b, rows_a, rows_b, hacc,
                   sem_ia, sem_ib, sem_ga, sem_gb):
        c = lax.axis_index("c")
        s = lax.axis_index("s")
        row0 = s * rows_pt
        pltpu.sync_copy(zeros_hbm.at[pl.ds(row0, rows_pt)],
                        hacc.at[pl.ds(row0, rows_pt)])
        if rem:
            @pl.when(s == 0)
            def _():
                pltpu.sync_copy(zeros_hbm.at[pl.ds(n_tiles * rows_pt, rem)],
                                hacc.at[pl.ds(n_tiles * rows_pt, rem)])
        plsc.subcore_barrier()

        def run(tref):
            def ifetch(k, ibuf, sem):
                pltpu.async_copy(eidx_hbm.at[s, k], ibuf, sem)

            def iwait(ibuf, sem):
                pltpu.make_async_copy(eidx_hbm.at[s, 0], ibuf, sem).wait()

            def gather(ibuf, rows, sem):
                pltpu.async_copy(tref.at[ibuf.at[0]], rows, sem)

            def gwait(ibuf, rows, sem):
                pltpu.make_async_copy(tref.at[ibuf.at[0]], rows, sem).wait()

            def scatter(ibuf, rows):
                pltpu.sync_copy(rows, hacc.at[ibuf.at[1]], add=True)

            def step(k, cur, nxt):
                (ibuf_c, sem_ic, rows_c, sem_gc) = cur
                (ibuf_n, sem_in, rows_n, sem_gn) = nxt
                iwait(ibuf_n, sem_in)
                gather(ibuf_n, rows_n, sem_gn)
                gwait(ibuf_c, rows_c, sem_gc)
                scatter(ibuf_c, rows_c)
                ifetch(k + 2, ibuf_c, sem_ic)

            buf_a = (ibuf_a, sem_ia, rows_a, sem_ga)
            buf_b = (ibuf_b, sem_ib, rows_b, sem_gb)

            ifetch(0, ibuf_a, sem_ia)
            iwait(ibuf_a, sem_ia)
            gather(ibuf_a, rows_a, sem_ga)
            ifetch(1, ibuf_b, sem_ib)

            def body(i2, carry):
                step(i2 * 2, buf_a, buf_b)
                step(i2 * 2 + 1, buf_b, buf_a)
                return carry
            lax.fori_loop(0, iters // 2 - 1, body, 0)
            iwait(ibuf_b, sem_ib)
            gather(ibuf_b, rows_b, sem_gb)
            gwait(ibuf_a, rows_a, sem_ga)
            scatter(ibuf_a, rows_a)
            gwait(ibuf_b, rows_b, sem_gb)
            scatter(ibuf_b, rows_b)

        @pl.when(c == 0)
        def _():
            run(t0_hbm)

        @pl.when(c == 1)
        def _():
            run(t1_hbm)

        plsc.subcore_barrier()
        pltpu.sync_copy(hacc.at[pl.ds(row0, rows_pt)],
                        out_hbm.at[c, pl.ds(row0, rows_pt)])
        if rem:
            @pl.when(s == 0)
            def _():
                pltpu.sync_copy(hacc.at[pl.ds(n_tiles * rows_pt, rem)],
                                out_hbm.at[c, pl.ds(n_tiles * rows_pt, rem)])

    return sc_scatter


def _combine_body(x_ref, h0_ref, h1_ref, nrm_ref, lwt_ref, b_ref, o_ref):
    x = x_ref[...]
    mx = jnp.dot(x, lwt_ref[...], preferred_element_type=jnp.float32)
    xn = _clipnorm(x)
    mxn = _clipnorm(mx)
    loop_msg = jnp.tanh(mxn / xn * _artanh(xn)) * mx / mxn
    h = jnp.concatenate([h0_ref[0], h1_ref[0]], axis=-1) * nrm_ref[...]
    h = _mobius_add(h, b_ref[...])
    h = _mobius_add(h, loop_msg)
    o_ref[...] = h


def _combine(x, h2, norm, lw_t, bias_row, tile_n):
    n, d = x.shape
    half = d // 2
    return pl.pallas_call(
        _combine_body,
        grid=(n // tile_n,),
        in_specs=[
            pl.BlockSpec((tile_n, d), lambda i: (i, 0)),
            pl.BlockSpec((1, tile_n, half), lambda i: (0, i, 0)),
            pl.BlockSpec((1, tile_n, half), lambda i: (1, i, 0)),
            pl.BlockSpec((tile_n, 1), lambda i: (i, 0)),
            pl.BlockSpec((d, d), lambda i: (0, 0)),
            pl.BlockSpec((1, d), lambda i: (0, 0)),
        ],
        out_specs=pl.BlockSpec((tile_n, d), lambda i: (i, 0)),
        out_shape=jax.ShapeDtypeStruct((n, d), jnp.float32),
    )(x, h2, h2, norm, lw_t, bias_row)


def kernel(x, edge_index, edge_type, norm, weight, loop_weight, bias):
    n, d = x.shape
    e = edge_index.shape[1]
    n_rel = weight.shape[0]
    n_blk = 4
    sub = d // n_blk
    half = d // 2

    src = edge_index[0].astype(jnp.int32)
    dst = edge_index[1].astype(jnp.int32)
    et = edge_type.astype(jnp.int32)
    gidx = et * n + src

    w4 = weight.reshape(n_rel, n_blk, sub, sub)
    eye = jnp.eye(n_blk, dtype=jnp.float32)
    w_bd = (w4[:, :, :, None, :] *
            eye[None, :, None, :, None]).reshape(n_rel, d, d)

    t0, t1 = _build_table(x, w_bd, tile_n=400)
    t0 = t0.reshape(n_rel * n, half)
    t1 = t1.reshape(n_rel * n, half)

    chunk = 128
    ep_raw = e // 16
    ep = -(-ep_raw // (2 * chunk)) * (2 * chunk)
    iters = ep // chunk
    pad = ep - ep_raw
    gidx_t = jnp.pad(gidx.reshape(16, ep_raw), ((0, 0), (0, pad)))
    dst_t = jnp.pad(dst.reshape(16, ep_raw), ((0, 0), (0, pad)),
                    constant_values=n)
    eidx = jnp.stack([gidx_t.reshape(16, iters, chunk),
                      dst_t.reshape(16, iters, chunk)], axis=2)
    zeros = jnp.zeros((n, half), jnp.float32)
    h2 = _make_sc_scatter(n, ep, half, chunk)(t0, t1, eidx, zeros)

    return _combine(x, h2, norm, loop_weight.T,
                    bias.reshape(1, d), tile_n=400)


# speedup vs baseline: 1.5897x; 1.0314x over previous
import functools

import jax
import jax.numpy as jnp
from jax import lax
from jax.experimental import pallas as pl
from jax.experimental.pallas import tpu as pltpu
from jax.experimental.pallas import tpu_sc as plsc

EPS = 1e-7


def _clipnorm(v):
    return jnp.clip(
        jnp.sqrt(jnp.sum(v * v, axis=-1, keepdims=True)), 1e-10, None)


def _artanh(v):
    v = jnp.clip(v, -1.0 + EPS, 1.0 - EPS)
    return 0.5 * jnp.log((1.0 + v) / (1.0 - v))


def _mobius_add(x, y):
    x2 = jnp.sum(x * x, axis=-1, keepdims=True)
    y2 = jnp.sum(y * y, axis=-1, keepdims=True)
    xy = jnp.sum(x * y, axis=-1, keepdims=True)
    num = (1.0 + 2.0 * xy + y2) * x + (1.0 - x2) * y
    den = 1.0 + 2.0 * xy + x2 * y2
    return num / jnp.clip(den, 1e-10, None)


def _table_body(x_ref, w_ref, out0_ref, out1_ref, *, n_rel, half):
    x = x_ref[...]
    xn = _clipnorm(x)
    node = x / xn * _artanh(xn)
    for r in range(n_rel):
        y = jnp.dot(node, w_ref[r], preferred_element_type=jnp.float32)
        yn = _clipnorm(y)
        z = jnp.tanh(yn) * y / yn
        out0_ref[r] = z[:, :half]
        out1_ref[r] = z[:, half:]


def _build_table(x, w_bd, tile_n):
    n, d = x.shape
    n_rel = w_bd.shape[0]
    half = d // 2
    body = functools.partial(_table_body, n_rel=n_rel, half=half)
    out_shape = jax.ShapeDtypeStruct((n_rel, n, half), jnp.float32)
    return pl.pallas_call(
        body,
        grid=(n // tile_n,),
        in_specs=[
            pl.BlockSpec((tile_n, d), lambda i: (i, 0)),
            pl.BlockSpec((n_rel, d, d), lambda i: (0, 0, 0)),
        ],
        out_specs=[
            pl.BlockSpec((n_rel, tile_n, half), lambda i: (0, i, 0)),
            pl.BlockSpec((n_rel, tile_n, half), lambda i: (0, i, 0)),
        ],
        out_shape=[out_shape, out_shape],
    )(x, w_bd)


def _make_sc_scatter(n, ep, half, chunk):
    n_tiles = 16
    iters = ep // chunk
    n_acc = n + 16
    rows_pt = (n // n_tiles) // 8 * 8
    rem = n - n_tiles * rows_pt
    mesh = plsc.VectorSubcoreMesh(core_axis_name="c", subcore_axis_name="s")

    @functools.partial(
        pl.kernel,
        out_type=jax.ShapeDtypeStruct((2, n, half), jnp.float32),
        mesh=mesh,
        scratch_types=[
            pltpu.VMEM((2, chunk), jnp.int32),
            pltpu.VMEM((2, chunk), jnp.int32),
            pltpu.VMEM((chunk, half), jnp.float32),
            pltpu.VMEM((chunk, half), jnp.float32),
            pltpu.VMEM_SHARED((n_acc, half), jnp.float32),
            pltpu.SemaphoreType.DMA,
            pltpu.SemaphoreType.DMA,
            pltpu.SemaphoreType.DMA,
            pltpu.SemaphoreType.DMA,
        ],
    )
    def sc_scatter(t0_hbm, t1_hbm, eidx_hbm, zeros_hbm, out_hbm,
                   ibuf_a, ibuf_b, rows_a, rows_b, hacc,
                   sem_ia, sem_ib, sem_ga, sem_gb):
        c = lax.axis_index("c")
        s = lax.axis_index("s")
        row0 = s * rows_pt
        pltpu.sync_copy(zeros_hbm.at[pl.ds(row0, rows_pt)],
                        hacc.at[pl.ds(row0, rows_pt)])
        if rem:
            @pl.when(s == 0)
            def _():
                pltpu.sync_copy(zeros_hbm.at[pl.ds(n_tiles * rows_pt, rem)],
                                hacc.at[pl.ds(n_tiles * rows_pt, rem)])
        plsc.subcore_barrier()

        def run(tref):
            def ifetch(k, ibuf, sem):
                pltpu.async_copy(eidx_hbm.at[s, k], ibuf, sem)

            def iwait(ibuf, sem):
                pltpu.make_async_copy(eidx_hbm.at[s, 0], ibuf, sem).wait()

            def gather(ibuf, rows, sem):
                pltpu.async_copy(tref.at[ibuf.at[0]], rows, sem)

            def gwait(ibuf, rows, sem):
                pltpu.make_async_copy(tref.at[ibuf.at[0]], rows, sem).wait()

            def scatter(ibuf, rows):
                pltpu.sync_copy(rows, hacc.at[ibuf.at[1]], add=True)

            def step(k, cur, nxt):
                (ibuf_c, sem_ic, rows_c, sem_gc) = cur
                (ibuf_n, sem_in, rows_n, sem_gn) = nxt
                iwait(ibuf_n, sem_in)
                gather(ibuf_n, rows_n, sem_gn)
                gwait(ibuf_c, rows_c, sem_gc)
                scatter(ibuf_c, rows_c)
                ifetch(k + 2, ibuf_c, sem_ic)

            buf_a = (ibuf_a, sem_ia, rows_a, sem_ga)
            buf_b = (ibuf_b, sem_ib, rows_b, sem_gb)

            ifetch(0, ibuf_a, sem_ia)
            iwait(ibuf_a, sem_ia)
            gather(ibuf_a, rows_a, sem_ga)
            ifetch(1, ibuf_b, sem_ib)

            def body(i2, carry):
                step(i2 * 2, buf_a, buf_b)
                step(i2 * 2 + 1, buf_b, buf_a)
                return carry
            lax.fori_loop(0, iters // 2 - 1, body, 0)
            iwait(ibuf_b, sem_ib)
            gather(ibuf_b, rows_b, sem_gb)
            gwait(ibuf_a, rows_a, sem_ga)
            scatter(ibuf_a, rows_a)
            gwait(ibuf_b, rows_b, sem_gb)
            scatter(ibuf_b, rows_b)

        @pl.when(c == 0)
        def _():
            run(t0_hbm)

        @pl.when(c == 1)
        def _():
            run(t1_hbm)

        plsc.subcore_barrier()
        pltpu.sync_copy(hacc.at[pl.ds(row0, rows_pt)],
                        out_hbm.at[c, pl.ds(row0, rows_pt)])
        if rem:
            @pl.when(s == 0)
            def _():
                pltpu.sync_copy(hacc.at[pl.ds(n_tiles * rows_pt, rem)],
                                out_hbm.at[c, pl.ds(n_tiles * rows_pt, rem)])

    return sc_scatter


def _combine_body(x_ref, h0_ref, h1_ref, nrm_ref, lwt_ref, b_ref, o_ref):
    x = x_ref[...]
    mx = jnp.dot(x, lwt_ref[...], preferred_element_type=jnp.float32)
    xn = _clipnorm(x)
    mxn = _clipnorm(mx)
    loop_msg = jnp.tanh(mxn / xn * _artanh(xn)) * mx / mxn
    h = jnp.concatenate([h0_ref[0], h1_ref[0]], axis=-1) * nrm_ref[...]
    h = _mobius_add(h, b_ref[...])
    h = _mobius_add(h, loop_msg)
    o_ref[...] = h


def _combine(x, h2, norm, lw_t, bias_row, tile_n):
    n, d = x.shape
    half = d // 2
    return pl.pallas_call(
        _combine_body,
        grid=(n // tile_n,),
        in_specs=[
            pl.BlockSpec((tile_n, d), lambda i: (i, 0)),
            pl.BlockSpec((1, tile_n, half), lambda i: (0, i, 0)),
            pl.BlockSpec((1, tile_n, half), lambda i: (1, i, 0)),
            pl.BlockSpec((tile_n, 1), lambda i: (i, 0)),
            pl.BlockSpec((d, d), lambda i: (0, 0)),
            pl.BlockSpec((1, d), lambda i: (0, 0)),
        ],
        out_specs=pl.BlockSpec((tile_n, d), lambda i: (i, 0)),
        out_shape=jax.ShapeDtypeStruct((n, d), jnp.float32),
    )(x, h2, h2, norm, lw_t, bias_row)


def kernel(x, edge_index, edge_type, norm, weight, loop_weight, bias):
    n, d = x.shape
    e = edge_index.shape[1]
    n_rel = weight.shape[0]
    n_blk = 4
    sub = d // n_blk
    half = d // 2

    src = edge_index[0].astype(jnp.int32)
    dst = edge_index[1].astype(jnp.int32)
    et = edge_type.astype(jnp.int32)
    gidx = et * n + src

    w4 = weight.reshape(n_rel, n_blk, sub, sub)
    eye = jnp.eye(n_blk, dtype=jnp.float32)
    w_bd = (w4[:, :, :, None, :] *
            eye[None, :, None, :, None]).reshape(n_rel, d, d)

    t0, t1 = _build_table(x, w_bd, tile_n=1000)
    t0 = t0.reshape(n_rel * n, half)
    t1 = t1.reshape(n_rel * n, half)

    chunk = 128
    ep_raw = e // 16
    ep = -(-ep_raw // (2 * chunk)) * (2 * chunk)
    iters = ep // chunk
    pad = ep - ep_raw
    gidx_t = jnp.pad(gidx.reshape(16, ep_raw), ((0, 0), (0, pad)))
    dst_t = jnp.pad(dst.reshape(16, ep_raw), ((0, 0), (0, pad)),
                    constant_values=n)
    eidx = jnp.stack([gidx_t.reshape(16, iters, chunk),
                      dst_t.reshape(16, iters, chunk)], axis=2)
    zeros = jnp.zeros((n, half), jnp.float32)
    h2 = _make_sc_scatter(n, ep, half, chunk)(t0, t1, eidx, zeros)

    return _combine(x, h2, norm, loop_weight.T,
                    bias.reshape(1, d), tile_n=1000)


# speedup vs baseline: 1.6005x; 1.0068x over previous
import functools

import jax
import jax.numpy as jnp
from jax import lax
from jax.experimental import pallas as pl
from jax.experimental.pallas import tpu as pltpu
from jax.experimental.pallas import tpu_sc as plsc

EPS = 1e-7


def _clipnorm(v):
    return jnp.clip(
        jnp.sqrt(jnp.sum(v * v, axis=-1, keepdims=True)), 1e-10, None)


def _artanh(v):
    v = jnp.clip(v, -1.0 + EPS, 1.0 - EPS)
    return 0.5 * jnp.log((1.0 + v) / (1.0 - v))


def _mobius_add(x, y):
    x2 = jnp.sum(x * x, axis=-1, keepdims=True)
    y2 = jnp.sum(y * y, axis=-1, keepdims=True)
    xy = jnp.sum(x * y, axis=-1, keepdims=True)
    num = (1.0 + 2.0 * xy + y2) * x + (1.0 - x2) * y
    den = 1.0 + 2.0 * xy + x2 * y2
    return num / jnp.clip(den, 1e-10, None)


def _table_body(x_ref, w_ref, out0_ref, out1_ref, *, n_rel, half):
    x = x_ref[...]
    xn = _clipnorm(x)
    node = x / xn * _artanh(xn)
    for r in range(n_rel):
        y = jnp.dot(node, w_ref[r], preferred_element_type=jnp.float32)
        yn = _clipnorm(y)
        z = jnp.tanh(yn) * y / yn
        out0_ref[r] = z[:, :half]
        out1_ref[r] = z[:, half:]


def _build_table(x, w_bd, tile_n):
    n, d = x.shape
    n_rel = w_bd.shape[0]
    half = d // 2
    body = functools.partial(_table_body, n_rel=n_rel, half=half)
    out_shape = jax.ShapeDtypeStruct((n_rel, n, half), jnp.float32)
    return pl.pallas_call(
        body,
        grid=(n // tile_n,),
        in_specs=[
            pl.BlockSpec((tile_n, d), lambda i: (i, 0)),
            pl.BlockSpec((n_rel, d, d), lambda i: (0, 0, 0)),
        ],
        out_specs=[
            pl.BlockSpec((n_rel, tile_n, half), lambda i: (0, i, 0)),
            pl.BlockSpec((n_rel, tile_n, half), lambda i: (0, i, 0)),
        ],
        out_shape=[out_shape, out_shape],
    )(x, w_bd)


def _make_sc_scatter(n, ep, half, chunk):
    n_tiles = 16
    iters = ep // chunk
    n_acc = n + 16
    rows_pt = (n // n_tiles) // 8 * 8
    rem = n - n_tiles * rows_pt
    mesh = plsc.VectorSubcoreMesh(core_axis_name="c", subcore_axis_name="s")

    @functools.partial(
        pl.kernel,
        out_type=jax.ShapeDtypeStruct((2, n, half), jnp.float32),
        mesh=mesh,
        scratch_types=[
            pltpu.VMEM((2, chunk), jnp.int32),
            pltpu.VMEM((2, chunk), jnp.int32),
            pltpu.VMEM((chunk, half), jnp.float32),
            pltpu.VMEM((chunk, half), jnp.float32),
            pltpu.VMEM_SHARED((n_acc, half), jnp.float32),
            pltpu.SemaphoreType.DMA,
            pltpu.SemaphoreType.DMA,
            pltpu.SemaphoreType.DMA,
            pltpu.SemaphoreType.DMA,
        ],
    )
    def sc_scatter(t0_hbm, t1_hbm, eidx_hbm, zeros_hbm, out_hbm,
                   ibuf_a, ibuf_b, rows_a, rows_b, hacc,
                   sem_ia, sem_ib, sem_ga, sem_gb):
        c = lax.axis_index("c")
        s = lax.axis_index("s")
        row0 = s * rows_pt
        pltpu.sync_copy(zeros_hbm.at[pl.ds(row0, rows_pt)],
                        hacc.at[pl.ds(row0, rows_pt)])
        if rem:
            @pl.when(s == 0)
            def _():
                pltpu.sync_copy(zeros_hbm.at[pl.ds(n_tiles * rows_pt, rem)],
                                hacc.at[pl.ds(n_tiles * rows_pt, rem)])
        plsc.subcore_barrier()

        def run(tref):
            def ifetch(k, ibuf, sem):
                pltpu.async_copy(eidx_hbm.at[s, k], ibuf, sem)

            def iwait(ibuf, sem):
                pltpu.make_async_copy(eidx_hbm.at[s, 0], ibuf, sem).wait()

            def gather(ibuf, rows, sem):
                pltpu.async_copy(tref.at[ibuf.at[0]], rows, sem)

            def gwait(ibuf, rows, sem):
                pltpu.make_async_copy(tref.at[ibuf.at[0]], rows, sem).wait()

            def scatter(ibuf, rows):
                pltpu.sync_copy(rows, hacc.at[ibuf.at[1]], add=True)

            def step(k, cur, nxt):
                (ibuf_c, sem_ic, rows_c, sem_gc) = cur
                (ibuf_n, sem_in, rows_n, sem_gn) = nxt
                iwait(ibuf_n, sem_in)
                gather(ibuf_n, rows_n, sem_gn)
                gwait(ibuf_c, rows_c, sem_gc)
                scatter(ibuf_c, rows_c)
                ifetch(k + 2, ibuf_c, sem_ic)

            buf_a = (ibuf_a, sem_ia, rows_a, sem_ga)
            buf_b = (ibuf_b, sem_ib, rows_b, sem_gb)

            ifetch(0, ibuf_a, sem_ia)
            iwait(ibuf_a, sem_ia)
            gather(ibuf_a, rows_a, sem_ga)
            ifetch(1, ibuf_b, sem_ib)

            def body(i2, carry):
                step(i2 * 2, buf_a, buf_b)
                step(i2 * 2 + 1, buf_b, buf_a)
                return carry
            lax.fori_loop(0, iters // 2 - 1, body, 0)
            iwait(ibuf_b, sem_ib)
            gather(ibuf_b, rows_b, sem_gb)
            gwait(ibuf_a, rows_a, sem_ga)
            scatter(ibuf_a, rows_a)
            gwait(ibuf_b, rows_b, sem_gb)
            scatter(ibuf_b, rows_b)

        @pl.when(c == 0)
        def _():
            run(t0_hbm)

        @pl.when(c == 1)
        def _():
            run(t1_hbm)

        plsc.subcore_barrier()
        pltpu.sync_copy(hacc.at[pl.ds(row0, rows_pt)],
                        out_hbm.at[c, pl.ds(row0, rows_pt)])
        if rem:
            @pl.when(s == 0)
            def _():
                pltpu.sync_copy(hacc.at[pl.ds(n_tiles * rows_pt, rem)],
                                out_hbm.at[c, pl.ds(n_tiles * rows_pt, rem)])

    return sc_scatter


def _combine_body(x_ref, h0_ref, h1_ref, nrm_ref, lwt_ref, b_ref, o_ref):
    x = x_ref[...]
    mx = jnp.dot(x, lwt_ref[...], preferred_element_type=jnp.float32)
    xn = _clipnorm(x)
    mxn = _clipnorm(mx)
    loop_msg = jnp.tanh(mxn / xn * _artanh(xn)) * mx / mxn
    h = jnp.concatenate([h0_ref[0], h1_ref[0]], axis=-1) * nrm_ref[...]
    h = _mobius_add(h, b_ref[...])
    h = _mobius_add(h, loop_msg)
    o_ref[...] = h


def _combine(x, h2, norm, lw_t, bias_row, tile_n):
    n, d = x.shape
    half = d // 2
    return pl.pallas_call(
        _combine_body,
        grid=(n // tile_n,),
        in_specs=[
            pl.BlockSpec((tile_n, d), lambda i: (i, 0)),
            pl.BlockSpec((1, tile_n, half), lambda i: (0, i, 0)),
            pl.BlockSpec((1, tile_n, half), lambda i: (1, i, 0)),
            pl.BlockSpec((tile_n, 1), lambda i: (i, 0)),
            pl.BlockSpec((d, d), lambda i: (0, 0)),
            pl.BlockSpec((1, d), lambda i: (0, 0)),
        ],
        out_specs=pl.BlockSpec((tile_n, d), lambda i: (i, 0)),
        out_shape=jax.ShapeDtypeStruct((n, d), jnp.float32),
    )(x, h2, h2, norm, lw_t, bias_row)


def kernel(x, edge_index, edge_type, norm, weight, loop_weight, bias):
    n, d = x.shape
    e = edge_index.shape[1]
    n_rel = weight.shape[0]
    n_blk = 4
    sub = d // n_blk
    half = d // 2

    src = edge_index[0].astype(jnp.int32)
    dst = edge_index[1].astype(jnp.int32)
    et = edge_type.astype(jnp.int32)
    gidx = et * n + src

    w4 = weight.reshape(n_rel, n_blk, sub, sub)
    eye = jnp.eye(n_blk, dtype=jnp.float32)
    w_bd = (w4[:, :, :, None, :] *
            eye[None, :, None, :, None]).reshape(n_rel, d, d)

    t0, t1 = _build_table(x, w_bd, tile_n=1000)
    t0 = t0.reshape(n_rel * n, half)
    t1 = t1.reshape(n_rel * n, half)

    chunk = 128
    ep_raw = e // 16
    ep = -(-ep_raw // (2 * chunk)) * (2 * chunk)
    iters = ep // chunk
    pad = ep - ep_raw
    gidx_t = jnp.pad(gidx.reshape(16, ep_raw), ((0, 0), (0, pad)))
    dst_t = jnp.pad(dst.reshape(16, ep_raw), ((0, 0), (0, pad)),
                    constant_values=n)
    eidx = jnp.stack([gidx_t.reshape(16, iters, chunk),
                      dst_t.reshape(16, iters, chunk)], axis=2)
    zeros = jnp.zeros((n, half), jnp.float32)
    h2 = _make_sc_scatter(n, ep, half, chunk)(t0, t1, eidx, zeros)

    return _combine(x, h2, norm, loop_weight.T,
                    bias.reshape(1, d), tile_n=2000)
